# Initial kernel scaffold; baseline (speedup 1.0000x reference)
import functools

import jax
import jax.numpy as jnp
from jax import lax
from jax.experimental import pallas as pl
from jax.experimental.pallas import tpu as pltpu
from jax.experimental.pallas import tpu_sc as plsc

EMBED_DIM = 16
NUM_CORES = 2
NUM_SUBCORES = 16
NUM_TILES = NUM_CORES * NUM_SUBCORES
IDX_W = 128
K = 16
CHUNK = K * IDX_W


def _scale_body(t_ref, o_ref):
  o_ref[...] = t_ref[...] * 4.0


def _scale_table(table):
  v, d = table.shape
  flat = table.reshape(v * d // 128, 128)
  rows = flat.shape[0]
  block = 1000
  scaled = pl.pallas_call(
      _scale_body,
      grid=(rows // block,),
      in_specs=[pl.BlockSpec((block, 128), lambda i: (i, 0))],
      out_specs=pl.BlockSpec((block, 128), lambda i: (i, 0)),
      out_shape=jax.ShapeDtypeStruct((rows, 128), jnp.float32),
  )(flat)
  return scaled.reshape(v, d)


@functools.partial(jax.jit, static_argnames=("n_rows_per_tile",))
def _sc_gather(idx2d, table, *, n_rows_per_tile):
  b = idx2d.shape[0] * IDX_W
  n_chunks = n_rows_per_tile // K
  mesh = plsc.VectorSubcoreMesh(
      core_axis_name="c", subcore_axis_name="s")

  @functools.partial(
      pl.kernel,
      out_type=jax.ShapeDtypeStruct((b, EMBED_DIM), jnp.float32),
      mesh=mesh,
      scratch_types=[
          pltpu.VMEM((K, IDX_W), jnp.int32),
          pltpu.VMEM((CHUNK, EMBED_DIM), jnp.float32),
          pltpu.SemaphoreType.DMA,
      ],
  )
  def gather_kernel(idx_hbm, table_hbm, out_hbm, idx_v, rows_v, sem):
    wid = lax.axis_index("s") * NUM_CORES + lax.axis_index("c")
    row_base = wid * n_rows_per_tile

    def chunk_body(i, carry):
      r0 = row_base + i * K
      pltpu.sync_copy(idx_hbm.at[pl.ds(r0, K)], idx_v)
      copies = []
      for j in range(K):
        copies.append(
            pltpu.async_copy(
                table_hbm.at[idx_v.at[j]],
                rows_v.at[pl.ds(j * IDX_W, IDX_W)],
                sem,
            ))
      for cp in copies:
        cp.wait()
      pltpu.sync_copy(rows_v, out_hbm.at[pl.ds(r0 * IDX_W, CHUNK)])
      return carry

    lax.fori_loop(0, n_chunks, chunk_body, 0)

  return gather_kernel(idx2d, table)


def kernel(tokens, table):
  n_tok = tokens.shape[0] * tokens.shape[1]
  idx2d = tokens.reshape(n_tok // IDX_W, IDX_W)
  scaled = _scale_table(table)
  n_rows_per_tile = idx2d.shape[0] // NUM_TILES
  out = _sc_gather(idx2d, scaled, n_rows_per_tile=n_rows_per_tile)
  return out.reshape(tokens.shape[0], tokens.shape[1], EMBED_DIM)


# speedup vs baseline: 2.3882x; 2.3882x over previous
import functools

import jax
import jax.numpy as jnp
from jax import lax
from jax.experimental import pallas as pl
from jax.experimental.pallas import tpu as pltpu
from jax.experimental.pallas import tpu_sc as plsc

EMBED_DIM = 16
NUM_CORES = 2
NUM_SUBCORES = 16
NUM_TILES = NUM_CORES * NUM_SUBCORES
IDX_W = 128
K = 16
CHUNK = K * IDX_W


def _scale_body(t_ref, o_ref):
  o_ref[...] = t_ref[...] * 4.0


def _scale_table(table):
  v, d = table.shape
  flat = table.reshape(v * d // 128, 128)
  rows = flat.shape[0]
  block = 1000
  scaled = pl.pallas_call(
      _scale_body,
      grid=(rows // block,),
      in_specs=[pl.BlockSpec((block, 128), lambda i: (i, 0))],
      out_specs=pl.BlockSpec((block, 128), lambda i: (i, 0)),
      out_shape=jax.ShapeDtypeStruct((rows, 128), jnp.float32),
  )(flat)
  return scaled.reshape(v, d)


@functools.partial(jax.jit, static_argnames=("n_rows_per_tile",))
def _sc_gather(idx2d, table, *, n_rows_per_tile):
  b = idx2d.shape[0] * IDX_W
  n_chunks = n_rows_per_tile // K
  mesh = plsc.VectorSubcoreMesh(
      core_axis_name="c", subcore_axis_name="s")

  @functools.partial(
      pl.kernel,
      out_type=jax.ShapeDtypeStruct((b, EMBED_DIM), jnp.float32),
      mesh=mesh,
      compiler_params=pltpu.CompilerParams(use_tc_tiling_on_sc=False),
      scratch_types=[
          pltpu.VMEM((K, IDX_W), jnp.int32),
          pltpu.VMEM((CHUNK, EMBED_DIM), jnp.float32),
          pltpu.SemaphoreType.DMA,
      ],
  )
  def gather_kernel(idx_hbm, table_hbm, out_hbm, idx_v, rows_v, sem):
    wid = lax.axis_index("s") * NUM_CORES + lax.axis_index("c")
    row_base = wid * n_rows_per_tile

    def chunk_body(i, carry):
      r0 = row_base + i * K
      pltpu.sync_copy(idx_hbm.at[pl.ds(r0, K)], idx_v)
      copies = []
      for j in range(K):
        copies.append(
            pltpu.async_copy(
                table_hbm.at[idx_v.at[j]],
                rows_v.at[pl.ds(j * IDX_W, IDX_W)],
                sem,
            ))
      for cp in copies:
        cp.wait()
      pltpu.sync_copy(rows_v, out_hbm.at[pl.ds(r0 * IDX_W, CHUNK)])
      return carry

    lax.fori_loop(0, n_chunks, chunk_body, 0)

  return gather_kernel(idx2d, table)


def kernel(tokens, table):
  n_tok = tokens.shape[0] * tokens.shape[1]
  idx2d = tokens.reshape(n_tok // IDX_W, IDX_W)
  scaled = _scale_table(table)
  n_rows_per_tile = idx2d.shape[0] // NUM_TILES
  out = _sc_gather(idx2d, scaled, n_rows_per_tile=n_rows_per_tile)
  return out.reshape(tokens.shape[0], tokens.shape[1], EMBED_DIM)


# speedup vs baseline: 2.4440x; 1.0234x over previous
import functools

import jax
import jax.numpy as jnp
from jax import lax
from jax.experimental import pallas as pl
from jax.experimental.pallas import tpu as pltpu
from jax.experimental.pallas import tpu_sc as plsc

EMBED_DIM = 16
NUM_CORES = 2
NUM_SUBCORES = 16
NUM_TILES = NUM_CORES * NUM_SUBCORES
IDX_W = 128
ROWS_PER_CHUNK = 16
TOK_PER_CHUNK = ROWS_PER_CHUNK * 200
STREAMS_PER_CHUNK = TOK_PER_CHUNK // IDX_W
UNROLL = 8


@functools.partial(jax.jit, static_argnames=("n_rows", "seq", "n_rows_per_tile"))
def _sc_embed(idx2d, table, *, n_rows, seq, n_rows_per_tile):
  mesh = plsc.VectorSubcoreMesh(core_axis_name="c", subcore_axis_name="s")
  n_chunks = n_rows_per_tile // ROWS_PER_CHUNK
  idx_rows_per_chunk = TOK_PER_CHUNK // IDX_W
  idx_rows_per_tile = n_rows_per_tile * seq // IDX_W

  @functools.partial(
      pl.kernel,
      out_type=jax.ShapeDtypeStruct((n_rows, seq, EMBED_DIM), jnp.float32),
      mesh=mesh,
      compiler_params=pltpu.CompilerParams(use_tc_tiling_on_sc=False),
      scratch_types=[
          pltpu.VMEM((idx_rows_per_chunk, IDX_W), jnp.int32),
          pltpu.VMEM((TOK_PER_CHUNK, EMBED_DIM), jnp.float32),
          pltpu.SemaphoreType.DMA,
      ],
  )
  def embed_kernel(idx_hbm, table_hbm, out_hbm, idx_v, rows_v, sem):
    wid = lax.axis_index("s") * NUM_CORES + lax.axis_index("c")
    tile_row0 = wid * n_rows_per_tile
    tile_idx_row0 = wid * idx_rows_per_tile

    def chunk_body(c, carry):
      pltpu.sync_copy(
          idx_hbm.at[pl.ds(tile_idx_row0 + c * idx_rows_per_chunk,
                           idx_rows_per_chunk)],
          idx_v)
      gathers = []
      for s in range(STREAMS_PER_CHUNK):
        gathers.append(
            pltpu.async_copy(
                table_hbm.at[idx_v.at[s]],
                rows_v.at[pl.ds(s * IDX_W, IDX_W)],
                sem,
            ))
      for g in gathers:
        g.wait()

      def scale_body(i, carry2):
        base = i * UNROLL
        for u in range(UNROLL):
          rows_v[base + u] = rows_v[base + u] * 4.0
        return carry2

      lax.fori_loop(0, TOK_PER_CHUNK // UNROLL, scale_body, 0)

      row0 = tile_row0 + c * ROWS_PER_CHUNK
      stores = []
      for r in range(ROWS_PER_CHUNK):
        stores.append(
            pltpu.async_copy(
                rows_v.at[pl.ds(r * seq, seq)],
                out_hbm.at[row0 + r],
                sem,
            ))
      for st in stores:
        st.wait()
      return carry

    lax.fori_loop(0, n_chunks, chunk_body, 0)

  return embed_kernel(idx2d, table)


def kernel(tokens, table):
  n_rows, seq = tokens.shape
  n_tok = n_rows * seq
  idx2d = tokens.reshape(n_tok // IDX_W, IDX_W)
  return _sc_embed(idx2d, table, n_rows=n_rows, seq=seq,
                   n_rows_per_tile=n_rows // NUM_TILES)


# speedup vs baseline: 2.9337x; 1.2004x over previous
import functools

import jax
import jax.numpy as jnp
from jax import lax
from jax.experimental import pallas as pl
from jax.experimental.pallas import tpu as pltpu
from jax.experimental.pallas import tpu_sc as plsc

EMBED_DIM = 16
NUM_CORES = 2
NUM_SUBCORES = 16
NUM_TILES = NUM_CORES * NUM_SUBCORES
IDX_W = 128
LANE = 16
I_PER_CHUNK = 16


@functools.partial(jax.jit, static_argnames=("n_rows", "seq"))
def _sc_embed(idx2d, table, *, n_rows, seq):
  mesh = plsc.VectorSubcoreMesh(core_axis_name="c", subcore_axis_name="s")
  tok_per_chunk = I_PER_CHUNK * seq
  streams_per_chunk = tok_per_chunk // IDX_W
  i_per_tile = n_rows // NUM_TILES
  n_chunks = i_per_tile // I_PER_CHUNK
  idx_rows_per_tile = i_per_tile * seq // IDX_W
  kd = EMBED_DIM // 8
  icols = n_rows // 128

  @functools.partial(
      pl.kernel,
      out_type=jax.ShapeDtypeStruct((seq, kd, icols, 8, 128), jnp.float32),
      mesh=mesh,
      compiler_params=pltpu.CompilerParams(
          use_tc_tiling_on_sc=False, needs_layout_passes=False),
      scratch_types=[
          pltpu.VMEM((streams_per_chunk, IDX_W), jnp.int32),
          pltpu.VMEM((tok_per_chunk, EMBED_DIM), jnp.float32),
          pltpu.VMEM((kd, seq, 8, LANE), jnp.float32),
          pltpu.SemaphoreType.DMA,
      ],
  )
  def embed_kernel(idx_hbm, table_hbm, out_hbm, idx_v, rows_v, tbuf, sem):
    wid = lax.axis_index("s") * NUM_CORES + lax.axis_index("c")
    i_tile0 = wid * i_per_tile
    idx_row0 = wid * idx_rows_per_tile

    def chunk_body(c, carry):
      pltpu.sync_copy(
          idx_hbm.at[pl.ds(idx_row0 + c * streams_per_chunk,
                           streams_per_chunk)],
          idx_v)
      gathers = []
      for s in range(streams_per_chunk):
        gathers.append(
            pltpu.async_copy(
                table_hbm.at[idx_v.at[s]],
                rows_v.at[pl.ds(s * IDX_W, IDX_W)],
                sem,
            ))
      for g in gathers:
        g.wait()

      row_iota = lax.iota(jnp.int32, LANE) * seq

      def tpose_body(j, carry2):
        idx0 = row_iota + j
        for k in range(EMBED_DIM):
          v = plsc.load_gather(
              rows_v, [idx0, jnp.full((LANE,), k, jnp.int32)])
          tbuf[k // 8, j, k % 8] = v * 4.0
        return carry2

      lax.fori_loop(0, seq, tpose_body, 0)

      i0 = i_tile0 + c * I_PER_CHUNK
      tc = i0 // 128
      ip = i0 % 128
      stores = []
      for tr in range(kd):
        stores.append(
            pltpu.async_copy(
                tbuf.at[tr],
                out_hbm.at[:, tr, tc, :, pl.ds(ip, LANE)],
                sem,
            ))
      for st in stores:
        st.wait()
      return carry

    lax.fori_loop(0, n_chunks, chunk_body, 0)

  return embed_kernel(idx2d, table)


def kernel(tokens, table):
  n_rows, seq = tokens.shape
  n_tok = n_rows * seq
  idx2d = tokens.reshape(n_tok // IDX_W, IDX_W)
  out5 = _sc_embed(idx2d, table, n_rows=n_rows, seq=seq)
  return out5.transpose(2, 4, 0, 1, 3).reshape(n_rows, seq, EMBED_DIM)


# speedup vs baseline: 4.9458x; 1.6859x over previous
import functools

import jax
import jax.numpy as jnp
from jax import lax
from jax.experimental import pallas as pl
from jax.experimental.pallas import tpu as pltpu
from jax.experimental.pallas import tpu_sc as plsc

EMBED_DIM = 16
NUM_CORES = 2
NUM_SUBCORES = 16
NUM_TILES = NUM_CORES * NUM_SUBCORES
IDX_W = 128
LANE = 16
I_PER_CHUNK = 16
JB = 25


@functools.partial(jax.jit, static_argnames=("n_rows", "seq"))
def _sc_embed(idx2d, table, *, n_rows, seq):
  mesh = plsc.VectorSubcoreMesh(core_axis_name="c", subcore_axis_name="s")
  tok_per_chunk = I_PER_CHUNK * seq
  streams = tok_per_chunk // IDX_W
  i_per_tile = n_rows // NUM_TILES
  n_chunks = i_per_tile // I_PER_CHUNK
  idx_rows_per_tile = i_per_tile * seq // IDX_W
  n_batches = seq // JB
  kd = EMBED_DIM // 8
  icols = n_rows // 128

  @functools.partial(
      pl.kernel,
      out_type=jax.ShapeDtypeStruct((seq, kd, icols, 8, 128), jnp.float32),
      mesh=mesh,
      compiler_params=pltpu.CompilerParams(
          use_tc_tiling_on_sc=False, needs_layout_passes=False),
      scratch_types=[
          pltpu.VMEM((2 * streams, IDX_W), jnp.int32),
          pltpu.VMEM((2 * tok_per_chunk, EMBED_DIM), jnp.float32),
          pltpu.VMEM((2 * kd, JB, 8, LANE), jnp.float32),
          pltpu.SemaphoreType.DMA,
          pltpu.SemaphoreType.DMA,
          pltpu.SemaphoreType.DMA,
          pltpu.SemaphoreType.DMA,
      ],
  )
  def embed_kernel(idx_hbm, table_hbm, out_hbm, idx_v, rows_v, tbuf,
                   sem_g0, sem_g1, sem_s0, sem_s1):
    wid = lax.axis_index("s") * NUM_CORES + lax.axis_index("c")
    i_tile0 = wid * i_per_tile
    idx_row0 = wid * idx_rows_per_tile
    sem_g = (sem_g0, sem_g1)
    sem_s = (sem_s0, sem_s1)
    kconsts = [jnp.full((LANE,), k, jnp.int32) for k in range(EMBED_DIM)]
    iota16 = lax.iota(jnp.int32, LANE)

    def fire_chunk(c, buf):
      pltpu.sync_copy(
          idx_hbm.at[pl.ds(idx_row0 + c * streams, streams)],
          idx_v.at[pl.ds(buf * streams, streams)])
      for s in range(streams):
        pltpu.async_copy(
            table_hbm.at[idx_v.at[buf * streams + s]],
            rows_v.at[pl.ds(buf * tok_per_chunk + s * IDX_W, IDX_W)],
            sem_g[buf])

    def drain_gathers(buf):
      for s in range(streams):
        pltpu.make_async_copy(
            table_hbm.at[idx_v.at[buf * streams + s]],
            rows_v.at[pl.ds(buf * tok_per_chunk + s * IDX_W, IDX_W)],
            sem_g[buf]).wait()

    def store_slice(c, b):
      i0 = i_tile0 + c * I_PER_CHUNK
      return out_hbm.at[pl.ds(b * JB, JB), 0, i0 // 128, :,
                        pl.ds(i0 % 128, LANE)]

    def drain_stores(c, b, t):
      for tr in range(kd):
        pltpu.make_async_copy(tbuf.at[t * kd + tr], store_slice(c, b),
                              sem_s[t]).wait()

    def do_chunk(c, p):
      q = 1 - p
      @pl.when(c + 1 < n_chunks)
      def _():
        fire_chunk(c + 1, q)

      drain_gathers(p)
      row_base = p * tok_per_chunk

      for b in range(n_batches):
        t = b % 2
        if b >= 2:
          drain_stores(c, b, t)
        else:
          @pl.when(c > 0)
          def _():
            drain_stores(c, b, t)
        j0 = b * JB

        @plsc.parallel_loop(j0, j0 + JB, 1, unroll=5)
        def _(j):
          idx0 = iota16 * seq + (row_base + j)
          vs = [plsc.load_gather(rows_v, [idx0, kconsts[k]])
                for k in range(EMBED_DIM)]
          for k in range(EMBED_DIM):
            tbuf[t * kd + k // 8, j - j0, k % 8] = vs[k] * 4.0

        i0 = i_tile0 + c * I_PER_CHUNK
        for tr in range(kd):
          pltpu.async_copy(
              tbuf.at[t * kd + tr],
              out_hbm.at[pl.ds(j0, JB), tr, i0 // 128, :,
                         pl.ds(i0 % 128, LANE)],
              sem_s[t])

    fire_chunk(0, 0)

    def super_body(c2, carry):
      do_chunk(c2 * 2, 0)
      do_chunk(c2 * 2 + 1, 1)
      return carry

    lax.fori_loop(0, n_chunks // 2, super_body, 0)
    drain_stores(n_chunks - 1, n_batches - 2, 0)
    drain_stores(n_chunks - 1, n_batches - 1, 1)

  return embed_kernel(idx2d, table)


def kernel(tokens, table):
  n_rows, seq = tokens.shape
  n_tok = n_rows * seq
  idx2d = tokens.reshape(n_tok // IDX_W, IDX_W)
  out5 = _sc_embed(idx2d, table, n_rows=n_rows, seq=seq)
  return out5.transpose(2, 4, 0, 1, 3).reshape(n_rows, seq, EMBED_DIM)


# speedup vs baseline: 7.9027x; 1.5979x over previous
import functools

import jax
import jax.numpy as jnp
from jax import lax
from jax.experimental import pallas as pl
from jax.experimental.pallas import tpu as pltpu
from jax.experimental.pallas import tpu_sc as plsc

EMBED_DIM = 16
NUM_CORES = 2
NUM_SUBCORES = 16
NUM_TILES = NUM_CORES * NUM_SUBCORES
IDX_W = 128
LANE = 16
I_PER_CHUNK = 16
JB = 20
ROW_W = 17


@functools.partial(jax.jit, static_argnames=("n_rows", "seq"))
def _sc_embed(idx2d, table, *, n_rows, seq):
  mesh = plsc.VectorSubcoreMesh(core_axis_name="c", subcore_axis_name="s")
  tok_per_chunk = I_PER_CHUNK * seq
  streams = tok_per_chunk // IDX_W
  i_per_tile = n_rows // NUM_TILES
  n_chunks = i_per_tile // I_PER_CHUNK
  idx_rows_per_tile = i_per_tile * seq // IDX_W
  n_batches = seq // JB
  kd = EMBED_DIM // 8
  icols = n_rows // 128

  @functools.partial(
      pl.kernel,
      out_type=jax.ShapeDtypeStruct((seq, kd, icols, 8, 128), jnp.float32),
      mesh=mesh,
      compiler_params=pltpu.CompilerParams(
          use_tc_tiling_on_sc=False, needs_layout_passes=False),
      scratch_types=[
          pltpu.VMEM((2 * streams, IDX_W), jnp.int32),
          pltpu.VMEM((2 * tok_per_chunk, EMBED_DIM), jnp.float32),
          pltpu.VMEM((2 * kd, JB, 8, ROW_W), jnp.float32),
          pltpu.SemaphoreType.DMA,
          pltpu.SemaphoreType.DMA,
          pltpu.SemaphoreType.DMA,
          pltpu.SemaphoreType.DMA,
      ],
  )
  def embed_kernel(idx_hbm, table_hbm, out_hbm, idx_v, rows_v, tbuf,
                   sem_g0, sem_g1, sem_s0, sem_s1):
    wid = lax.axis_index("s") * NUM_CORES + lax.axis_index("c")
    i_tile0 = wid * i_per_tile
    idx_row0 = wid * idx_rows_per_tile
    sem_g = (sem_g0, sem_g1)
    sem_s = (sem_s0, sem_s1)
    kconsts = [jnp.full((LANE,), k, jnp.int32) for k in range(EMBED_DIM)]
    iota16 = lax.iota(jnp.int32, LANE)

    def fire_chunk(c, buf):
      pltpu.sync_copy(
          idx_hbm.at[pl.ds(idx_row0 + c * streams, streams)],
          idx_v.at[pl.ds(buf * streams, streams)])
      for s in range(streams):
        pltpu.async_copy(
            table_hbm.at[idx_v.at[buf * streams + s]],
            rows_v.at[pl.ds(buf * tok_per_chunk + s * IDX_W, IDX_W)],
            sem_g[buf])

    def drain_gathers(buf):
      for s in range(streams):
        pltpu.make_async_copy(
            table_hbm.at[idx_v.at[buf * streams + s]],
            rows_v.at[pl.ds(buf * tok_per_chunk + s * IDX_W, IDX_W)],
            sem_g[buf]).wait()

    def store_slice(c, b):
      i0 = i_tile0 + c * I_PER_CHUNK
      return out_hbm.at[pl.ds(b * JB, JB), 0, i0 // 128, :,
                        pl.ds(i0 % 128, LANE)]

    def drain_stores(c, b, t):
      for tr in range(kd):
        pltpu.make_async_copy(
            tbuf.at[t * kd + tr, pl.ds(0, JB), pl.ds(0, 8), pl.ds(0, LANE)],
            store_slice(c, b), sem_s[t]).wait()

    def do_chunk(c, p):
      q = 1 - p
      @pl.when(c + 1 < n_chunks)
      def _():
        fire_chunk(c + 1, q)

      drain_gathers(p)
      row_base = p * tok_per_chunk

      for b in range(n_batches):
        t = b % 2
        if b >= 2:
          drain_stores(c, b, t)
        else:
          @pl.when(c > 0)
          def _():
            drain_stores(c, b, t)
        j0 = b * JB

        d0c = t * kd + iota16 // 8
        d2c = iota16 % 8

        @plsc.parallel_loop(j0, j0 + JB, 1, unroll=4)
        def _(j):
          jj = jnp.full((LANE,), j - j0, jnp.int32)
          vals = [rows_v[row_base + ip * seq + j] * 4.0
                  for ip in range(I_PER_CHUNK)]
          for ip in range(I_PER_CHUNK):
            plsc.store_scatter(tbuf, [d0c, jj, d2c, kconsts[ip]], vals[ip])

        i0 = i_tile0 + c * I_PER_CHUNK
        for tr in range(kd):
          pltpu.async_copy(
              tbuf.at[t * kd + tr, pl.ds(0, JB), pl.ds(0, 8),
                      pl.ds(0, LANE)],
              out_hbm.at[pl.ds(j0, JB), tr, i0 // 128, :,
                         pl.ds(i0 % 128, LANE)],
              sem_s[t])

    fire_chunk(0, 0)

    def super_body(c2, carry):
      do_chunk(c2 * 2, 0)
      do_chunk(c2 * 2 + 1, 1)
      return carry

    lax.fori_loop(0, n_chunks // 2, super_body, 0)
    drain_stores(n_chunks - 1, n_batches - 2, 0)
    drain_stores(n_chunks - 1, n_batches - 1, 1)

  return embed_kernel(idx2d, table)


def kernel(tokens, table):
  n_rows, seq = tokens.shape
  n_tok = n_rows * seq
  idx2d = tokens.reshape(n_tok // IDX_W, IDX_W)
  out5 = _sc_embed(idx2d, table, n_rows=n_rows, seq=seq)
  return out5.transpose(2, 4, 0, 1, 3).reshape(n_rows, seq, EMBED_DIM)


# speedup vs baseline: 7.9063x; 1.0004x over previous
import functools

import jax
import jax.numpy as jnp
from jax import lax
from jax.experimental import pallas as pl
from jax.experimental.pallas import tpu as pltpu
from jax.experimental.pallas import tpu_sc as plsc

EMBED_DIM = 16
NUM_CORES = 2
NUM_SUBCORES = 16
NUM_TILES = NUM_CORES * NUM_SUBCORES
IDX_W = 128
LANE = 16
I_PER_CHUNK = 16
JB = 25
ROW_W = 17


@functools.partial(jax.jit, static_argnames=("n_rows", "seq"))
def _sc_embed(idx2d, table, *, n_rows, seq):
  mesh = plsc.VectorSubcoreMesh(core_axis_name="c", subcore_axis_name="s")
  tok_per_chunk = I_PER_CHUNK * seq
  streams = tok_per_chunk // IDX_W
  i_per_tile = n_rows // NUM_TILES
  n_chunks = i_per_tile // I_PER_CHUNK
  idx_rows_per_tile = i_per_tile * seq // IDX_W
  n_batches = seq // JB
  kd = EMBED_DIM // 8
  icols = n_rows // 128

  @functools.partial(
      pl.kernel,
      out_type=jax.ShapeDtypeStruct((seq, kd, icols, 8, 128), jnp.float32),
      mesh=mesh,
      compiler_params=pltpu.CompilerParams(
          use_tc_tiling_on_sc=False, needs_layout_passes=False),
      scratch_types=[
          pltpu.VMEM((2 * streams, IDX_W), jnp.int32),
          pltpu.VMEM((2 * tok_per_chunk, EMBED_DIM), jnp.float32),
          pltpu.VMEM((2 * kd, JB, 8, ROW_W), jnp.float32),
          pltpu.SemaphoreType.DMA,
          pltpu.SemaphoreType.DMA,
          pltpu.SemaphoreType.DMA,
          pltpu.SemaphoreType.DMA,
      ],
  )
  def embed_kernel(idx_hbm, table_hbm, out_hbm, idx_v, rows_v, tbuf,
                   sem_g0, sem_g1, sem_s0, sem_s1):
    wid = lax.axis_index("s") * NUM_CORES + lax.axis_index("c")
    i_tile0 = wid * i_per_tile
    idx_row0 = wid * idx_rows_per_tile
    sem_g = (sem_g0, sem_g1)
    sem_s = (sem_s0, sem_s1)
    kconsts = [jnp.full((LANE,), k, jnp.int32) for k in range(EMBED_DIM)]
    iota16 = lax.iota(jnp.int32, LANE)

    def fire_chunk(c, buf):
      pltpu.sync_copy(
          idx_hbm.at[pl.ds(idx_row0 + c * streams, streams)],
          idx_v.at[pl.ds(buf * streams, streams)])
      for s in range(streams):
        pltpu.async_copy(
            table_hbm.at[idx_v.at[buf * streams + s]],
            rows_v.at[pl.ds(buf * tok_per_chunk + s * IDX_W, IDX_W)],
            sem_g[buf])

    def drain_gathers(buf):
      for s in range(streams):
        pltpu.make_async_copy(
            table_hbm.at[idx_v.at[buf * streams + s]],
            rows_v.at[pl.ds(buf * tok_per_chunk + s * IDX_W, IDX_W)],
            sem_g[buf]).wait()

    def store_slice(c, b):
      i0 = i_tile0 + c * I_PER_CHUNK
      return out_hbm.at[pl.ds(b * JB, JB), 0, i0 // 128, :,
                        pl.ds(i0 % 128, LANE)]

    def drain_stores(c, b, t):
      for tr in range(kd):
        pltpu.make_async_copy(
            tbuf.at[t * kd + tr, pl.ds(0, JB), pl.ds(0, 8), pl.ds(0, LANE)],
            store_slice(c, b), sem_s[t]).wait()

    def do_chunk(c, p):
      q = 1 - p
      @pl.when(c + 1 < n_chunks)
      def _():
        fire_chunk(c + 1, q)

      drain_gathers(p)
      row_base = p * tok_per_chunk

      for b in range(n_batches):
        t = b % 2
        if b >= 2:
          drain_stores(c, b, t)
        else:
          @pl.when(c > 0)
          def _():
            drain_stores(c, b, t)
        j0 = b * JB

        d0c = t * kd + iota16 // 8
        d2c = iota16 % 8

        @plsc.parallel_loop(j0, j0 + JB, 1, unroll=5)
        def _(j):
          jj = jnp.full((LANE,), j - j0, jnp.int32)
          vals = [rows_v[row_base + ip * seq + j] * 4.0
                  for ip in range(I_PER_CHUNK)]
          for ip in range(I_PER_CHUNK):
            plsc.store_scatter(tbuf, [d0c, jj, d2c, kconsts[ip]], vals[ip])

        i0 = i_tile0 + c * I_PER_CHUNK
        for tr in range(kd):
          pltpu.async_copy(
              tbuf.at[t * kd + tr, pl.ds(0, JB), pl.ds(0, 8),
                      pl.ds(0, LANE)],
              out_hbm.at[pl.ds(j0, JB), tr, i0 // 128, :,
                         pl.ds(i0 % 128, LANE)],
              sem_s[t])

    fire_chunk(0, 0)

    def super_body(c2, carry):
      do_chunk(c2 * 2, 0)
      do_chunk(c2 * 2 + 1, 1)
      return carry

    lax.fori_loop(0, n_chunks // 2, super_body, 0)
    drain_stores(n_chunks - 1, n_batches - 2, 0)
    drain_stores(n_chunks - 1, n_batches - 1, 1)

  return embed_kernel(idx2d, table)


def kernel(tokens, table):
  n_rows, seq = tokens.shape
  n_tok = n_rows * seq
  idx2d = tokens.reshape(n_tok // IDX_W, IDX_W)
  out5 = _sc_embed(idx2d, table, n_rows=n_rows, seq=seq)
  return out5.transpose(2, 4, 0, 1, 3).reshape(n_rows, seq, EMBED_DIM)
